# baseline (speedup 1.0000x reference)
import functools
import math

import jax
import jax.numpy as jnp
from jax.experimental import pallas as pl


def _masked_attn_kernel(mask_ref, q_ref, k_ref, v_ref, o_ref, *, scale, H, D):
    mask = mask_ref[...]
    for h in range(H):
        sl = slice(h * D, (h + 1) * D)
        q = (q_ref[:, sl] * scale).astype(jnp.bfloat16)
        k = k_ref[:, sl].astype(jnp.bfloat16)
        v = v_ref[:, sl].astype(jnp.bfloat16)
        s = jax.lax.dot_general(
            q, k, (((1,), (1,)), ((), ())), preferred_element_type=jnp.float32
        )
        s = jnp.where(mask, s, -1e30)
        m = jnp.max(s, axis=1, keepdims=True)
        m = jnp.maximum(m, -1e29)
        e = jnp.exp(s - m)
        denom = jnp.sum(e, axis=1, keepdims=True)
        r = 1.0 / jnp.maximum(denom, 1.0)
        o = jax.lax.dot_general(
            e.astype(jnp.bfloat16), v, (((1,), (0,)), ((), ())),
            preferred_element_type=jnp.float32,
        )
        o_ref[:, sl] = o * r


def kernel(q, k, v, pattern_mask):
    B, S, H, D = q.shape
    q2 = q.reshape(B * S, H * D)
    k2 = k.reshape(B * S, H * D)
    v2 = v.reshape(B * S, H * D)

    RB = 128
    out = pl.pallas_call(
        functools.partial(
            _masked_attn_kernel, scale=1.0 / math.sqrt(D), H=H, D=D
        ),
        grid=(S // RB,),
        in_specs=[
            pl.BlockSpec((RB, S), lambda i: (i, 0)),
            pl.BlockSpec((RB, H * D), lambda i: (i, 0)),
            pl.BlockSpec((S, H * D), lambda i: (0, 0)),
            pl.BlockSpec((S, H * D), lambda i: (0, 0)),
        ],
        out_specs=pl.BlockSpec((RB, H * D), lambda i: (i, 0)),
        out_shape=jax.ShapeDtypeStruct((B * S, H * D), jnp.float32),
    )(pattern_mask, q2, k2, v2)

    return out.reshape(B, S, H, D)


# speedup vs baseline: 1.2613x; 1.2613x over previous
import functools
import math

import jax
import jax.numpy as jnp
from jax.experimental import pallas as pl


def _masked_attn_kernel(mask_ref, q_ref, k_ref, v_ref, o_ref, *, scale, H, D):
    mask = mask_ref[...]
    for h in range(H):
        sl = slice(h * D, (h + 1) * D)
        q = (q_ref[:, sl] * scale).astype(jnp.bfloat16)
        k = k_ref[:, sl].astype(jnp.bfloat16)
        v = v_ref[:, sl].astype(jnp.bfloat16)
        s = jax.lax.dot_general(
            q, k, (((1,), (1,)), ((), ())), preferred_element_type=jnp.float32
        )
        s = jnp.where(mask, s, -1e30)
        m = jnp.max(s, axis=1, keepdims=True)
        m = jnp.maximum(m, -1e29)
        e = jnp.exp(s - m)
        denom = jnp.sum(e, axis=1, keepdims=True)
        r = 1.0 / jnp.maximum(denom, 1.0)
        o = jax.lax.dot_general(
            e.astype(jnp.bfloat16), v, (((1,), (0,)), ((), ())),
            preferred_element_type=jnp.float32,
        )
        o_ref[:, sl] = o * r


def kernel(q, k, v, pattern_mask):
    B, S, H, D = q.shape
    q2 = q.reshape(B * S, H * D)
    k2 = k.reshape(B * S, H * D)
    v2 = v.reshape(B * S, H * D)

    HQ = 128
    out = pl.pallas_call(
        functools.partial(
            _masked_attn_kernel, scale=1.0 / math.sqrt(D), H=HQ // D, D=D
        ),
        grid=(H * D // HQ,),
        in_specs=[
            pl.BlockSpec((S, S), lambda g: (0, 0)),
            pl.BlockSpec((B * S, HQ), lambda g: (0, g)),
            pl.BlockSpec((B * S, HQ), lambda g: (0, g)),
            pl.BlockSpec((B * S, HQ), lambda g: (0, g)),
        ],
        out_specs=pl.BlockSpec((B * S, HQ), lambda g: (0, g)),
        out_shape=jax.ShapeDtypeStruct((B * S, H * D), jnp.float32),
    )(pattern_mask, q2, k2, v2)

    return out.reshape(B, S, H, D)
